# baseline (speedup 1.0000x reference)
import functools

import jax
import jax.numpy as jnp
from jax import lax
from jax.experimental import pallas as pl
from jax.experimental.pallas import tpu as pltpu
from jax.experimental.pallas import tpu_sc as plsc

B = 4096
L = 200
D = 64
H = 100
W = D // 2
NW = 32
BPW = B // NW
NBUF = 4
NC = 2

PERM = [32 * (c // 32) + 16 * ((c % 32) % 2) + ((c % 32) // 2)
        for c in range(D)]

_mesh = plsc.VectorSubcoreMesh(core_axis_name="c", subcore_axis_name="s")


@functools.partial(
    pl.kernel,
    mesh=_mesh,
    out_type=jax.ShapeDtypeStruct((B, D), jnp.float32),
    compiler_params=pltpu.CompilerParams(use_tc_tiling_on_sc=False),
    scratch_types=[
        pltpu.VMEM((2 * BPW, H), jnp.int32),
        pltpu.VMEM((BPW, 16), jnp.float32),
        pltpu.VMEM((NBUF, 2 * H, W), jnp.int32),
        pltpu.VMEM((BPW, D), jnp.float32),
        pltpu.SemaphoreType.DMA,
        pltpu.SemaphoreType.DMA,
        pltpu.SemaphoreType.DMA,
        pltpu.SemaphoreType.DMA,
    ],
)
def _bow_sc(table, idx_hbm, recip_hbm, out_hbm, idx_v, recip_v, bufs, out_v,
            sem0, sem1, sem2, sem3):
    sems = (sem0, sem1, sem2, sem3)
    wid = lax.axis_index("s") * NC + lax.axis_index("c")
    pltpu.sync_copy(idx_hbm.at[pl.ds(wid * (2 * BPW), 2 * BPW)], idx_v)
    pltpu.sync_copy(recip_hbm.at[pl.ds(wid * BPW, BPW)], recip_v)

    def fire(bag, k):
        pltpu.make_async_copy(
            table.at[idx_v.at[2 * bag]],
            bufs.at[k, pl.ds(0, H)], sems[k]).start()
        pltpu.make_async_copy(
            table.at[idx_v.at[2 * bag + 1]],
            bufs.at[k, pl.ds(H, H)], sems[k]).start()

    def wait(k):
        pltpu.make_async_copy(
            table.at[idx_v.at[0]], bufs.at[k, pl.ds(0, H)], sems[k]).wait()
        pltpu.make_async_copy(
            table.at[idx_v.at[0]], bufs.at[k, pl.ds(H, H)], sems[k]).wait()

    def reduce_bag(bag, k):
        mask = jnp.full((16,), -65536, jnp.int32)

        def body(r, acc):
            a0, a1, a2, a3 = acc
            w0 = bufs[k, r, pl.ds(0, 16)]
            w1 = bufs[k, r, pl.ds(16, 16)]
            a0 = a0 + lax.bitcast_convert_type(
                lax.shift_left(w0, 16), jnp.float32)
            a1 = a1 + lax.bitcast_convert_type(w0 & mask, jnp.float32)
            a2 = a2 + lax.bitcast_convert_type(
                lax.shift_left(w1, 16), jnp.float32)
            a3 = a3 + lax.bitcast_convert_type(w1 & mask, jnp.float32)
            return a0, a1, a2, a3

        z = jnp.zeros((16,), jnp.float32)
        a0, a1, a2, a3 = lax.fori_loop(0, 2 * H, body, (z, z, z, z),
                                       unroll=4)
        rc = recip_v[bag, pl.ds(0, 16)]
        out_v[bag, pl.ds(0, 16)] = a0 * rc
        out_v[bag, pl.ds(16, 16)] = a1 * rc
        out_v[bag, pl.ds(32, 16)] = a2 * rc
        out_v[bag, pl.ds(48, 16)] = a3 * rc

    for k in range(NBUF - 1):
        fire(jnp.int32(k), k)

    def outer(g, carry):
        base = g * NBUF
        for k in range(NBUF):
            bag = base + k
            wait(k)
            nxt = bag + (NBUF - 1)

            @pl.when(nxt < BPW)
            def _():
                fire(nxt, (k + NBUF - 1) % NBUF)

            reduce_bag(bag, k)
        return carry

    lax.fori_loop(0, BPW // NBUF, outer, 0)
    pltpu.sync_copy(out_v, out_hbm.at[pl.ds(wid * BPW, BPW)])


def kernel(x, length, emb_weight):
    idx = x.astype(jnp.int32).reshape(2 * B, H)
    recip = jnp.broadcast_to((1.0 / length.astype(jnp.float32))[:, None], (B, 16))
    emb_bf = emb_weight[:, jnp.array(PERM)].astype(jnp.bfloat16)
    packed = jax.lax.bitcast_convert_type(
        emb_bf.reshape(emb_weight.shape[0], W, 2), jnp.int32)
    return _bow_sc(packed, idx, recip)


# speedup vs baseline: 1.2561x; 1.2561x over previous
import functools

import jax
import jax.numpy as jnp
from jax import lax
from jax.experimental import pallas as pl
from jax.experimental.pallas import tpu as pltpu
from jax.experimental.pallas import tpu_sc as plsc

B = 4096
L = 200
D = 64
H = 100
W = D // 2
NW = 32
BPW = B // NW
NBUF = 4
NC = 2

_mesh = plsc.VectorSubcoreMesh(core_axis_name="c", subcore_axis_name="s")


@functools.partial(
    pl.kernel,
    mesh=_mesh,
    out_type=jax.ShapeDtypeStruct((B, D), jnp.float32),
    compiler_params=pltpu.CompilerParams(use_tc_tiling_on_sc=False),
    scratch_types=[
        pltpu.VMEM((2 * BPW, H), jnp.int32),
        pltpu.VMEM((BPW, 16), jnp.float32),
        pltpu.VMEM((NBUF, 2 * H, W), jnp.int32),
        pltpu.VMEM((BPW, D), jnp.float32),
        pltpu.SemaphoreType.DMA,
        pltpu.SemaphoreType.DMA,
        pltpu.SemaphoreType.DMA,
        pltpu.SemaphoreType.DMA,
    ],
)
def _bow_sc(table, idx_hbm, recip_hbm, out_hbm, idx_v, recip_v, bufs, out_v,
            sem0, sem1, sem2, sem3):
    sems = (sem0, sem1, sem2, sem3)
    wid = lax.axis_index("s") * NC + lax.axis_index("c")
    pltpu.sync_copy(idx_hbm.at[pl.ds(wid * (2 * BPW), 2 * BPW)], idx_v)
    pltpu.sync_copy(recip_hbm.at[pl.ds(wid * BPW, BPW)], recip_v)

    def fire(bag, k):
        pltpu.make_async_copy(
            table.at[idx_v.at[2 * bag]],
            bufs.at[k, pl.ds(0, H)], sems[k]).start()
        pltpu.make_async_copy(
            table.at[idx_v.at[2 * bag + 1]],
            bufs.at[k, pl.ds(H, H)], sems[k]).start()

    def wait(k):
        pltpu.make_async_copy(
            table.at[idx_v.at[0]], bufs.at[k, pl.ds(0, H)], sems[k]).wait()
        pltpu.make_async_copy(
            table.at[idx_v.at[0]], bufs.at[k, pl.ds(H, H)], sems[k]).wait()

    def reduce_bag(bag, k):
        mask = jnp.full((16,), -65536, jnp.int32)

        def body(r, acc):
            a0, a1, a2, a3 = acc
            w0 = bufs[k, r, pl.ds(0, 16)]
            w1 = bufs[k, r, pl.ds(16, 16)]
            a0 = a0 + lax.bitcast_convert_type(
                lax.shift_left(w0, 16), jnp.float32)
            a1 = a1 + lax.bitcast_convert_type(w0 & mask, jnp.float32)
            a2 = a2 + lax.bitcast_convert_type(
                lax.shift_left(w1, 16), jnp.float32)
            a3 = a3 + lax.bitcast_convert_type(w1 & mask, jnp.float32)
            return a0, a1, a2, a3

        z = jnp.zeros((16,), jnp.float32)
        a0, a1, a2, a3 = lax.fori_loop(0, 2 * H, body, (z, z, z, z),
                                       unroll=4)
        rc = recip_v[bag, pl.ds(0, 16)]
        out_v[bag, pl.ds(0, 16)] = a0 * rc
        out_v[bag, pl.ds(16, 16)] = a1 * rc
        out_v[bag, pl.ds(32, 16)] = a2 * rc
        out_v[bag, pl.ds(48, 16)] = a3 * rc

    for k in range(NBUF - 1):
        fire(jnp.int32(k), k)

    def outer(g, carry):
        base = g * NBUF
        for k in range(NBUF):
            bag = base + k
            wait(k)
            nxt = bag + (NBUF - 1)

            @pl.when(nxt < BPW)
            def _():
                fire(nxt, (k + NBUF - 1) % NBUF)

            reduce_bag(bag, k)
        return carry

    lax.fori_loop(0, BPW // NBUF, outer, 0)
    pltpu.sync_copy(out_v, out_hbm.at[pl.ds(wid * BPW, BPW)])


def kernel(x, length, emb_weight):
    idx = x.astype(jnp.int32).reshape(2 * B, H)
    recip = jnp.broadcast_to((1.0 / length.astype(jnp.float32))[:, None], (B, 16))
    emb_bf = emb_weight.astype(jnp.bfloat16)
    packed = jax.lax.bitcast_convert_type(
        emb_bf.reshape(emb_weight.shape[0], W, 2), jnp.int32)
    out = _bow_sc(packed, idx, recip)
    return out.reshape(B, 2, 2, 16).transpose(0, 1, 3, 2).reshape(B, D)


# speedup vs baseline: 3.2080x; 2.5539x over previous
import functools

import jax
import jax.numpy as jnp
from jax import lax
from jax.experimental import pallas as pl
from jax.experimental.pallas import tpu as pltpu
from jax.experimental.pallas import tpu_sc as plsc

B = 4096
L = 200
D = 64
H = 100
NW = 32
BPW = B // NW
NBUF = 4
NC = 2

_mesh = plsc.VectorSubcoreMesh(core_axis_name="c", subcore_axis_name="s")


@functools.partial(
    pl.kernel,
    mesh=_mesh,
    out_type=jax.ShapeDtypeStruct((B, D), jnp.float32),
    compiler_params=pltpu.CompilerParams(use_tc_tiling_on_sc=False),
    scratch_types=[
        pltpu.VMEM((2 * BPW, H), jnp.int32),
        pltpu.VMEM((BPW, 16), jnp.float32),
        pltpu.VMEM((NBUF, 2 * H, D), jnp.float32),
        pltpu.VMEM((BPW, D), jnp.float32),
        pltpu.SemaphoreType.DMA,
        pltpu.SemaphoreType.DMA,
        pltpu.SemaphoreType.DMA,
        pltpu.SemaphoreType.DMA,
    ],
)
def _bow_sc(table, idx_hbm, recip_hbm, out_hbm, idx_v, recip_v, bufs, out_v,
            sem0, sem1, sem2, sem3):
    sems = (sem0, sem1, sem2, sem3)
    wid = lax.axis_index("s") * NC + lax.axis_index("c")
    pltpu.sync_copy(idx_hbm.at[pl.ds(wid * (2 * BPW), 2 * BPW)], idx_v)
    pltpu.sync_copy(recip_hbm.at[pl.ds(wid * BPW, BPW)], recip_v)

    def fire(bag, k):
        pltpu.make_async_copy(
            table.at[idx_v.at[2 * bag]],
            bufs.at[k, pl.ds(0, H)], sems[k]).start()
        pltpu.make_async_copy(
            table.at[idx_v.at[2 * bag + 1]],
            bufs.at[k, pl.ds(H, H)], sems[k]).start()

    def wait(k):
        pltpu.make_async_copy(
            table.at[idx_v.at[0]], bufs.at[k, pl.ds(0, H)], sems[k]).wait()
        pltpu.make_async_copy(
            table.at[idx_v.at[0]], bufs.at[k, pl.ds(H, H)], sems[k]).wait()

    def reduce_bag(bag, k):
        def body(r, acc):
            a0, a1, a2, a3 = acc
            a0 = a0 + bufs[k, r, pl.ds(0, 16)]
            a1 = a1 + bufs[k, r, pl.ds(16, 16)]
            a2 = a2 + bufs[k, r, pl.ds(32, 16)]
            a3 = a3 + bufs[k, r, pl.ds(48, 16)]
            return a0, a1, a2, a3

        z = jnp.zeros((16,), jnp.float32)
        a0, a1, a2, a3 = lax.fori_loop(0, 2 * H, body, (z, z, z, z),
                                       unroll=8)
        rc = recip_v[bag, pl.ds(0, 16)]
        out_v[bag, pl.ds(0, 16)] = a0 * rc
        out_v[bag, pl.ds(16, 16)] = a1 * rc
        out_v[bag, pl.ds(32, 16)] = a2 * rc
        out_v[bag, pl.ds(48, 16)] = a3 * rc

    for k in range(NBUF - 1):
        fire(jnp.int32(k), k)

    def outer(g, carry):
        base = g * NBUF
        for k in range(NBUF):
            bag = base + k
            wait(k)
            nxt = bag + (NBUF - 1)

            @pl.when(nxt < BPW)
            def _():
                fire(nxt, (k + NBUF - 1) % NBUF)

            reduce_bag(bag, k)
        return carry

    lax.fori_loop(0, BPW // NBUF, outer, 0)
    pltpu.sync_copy(out_v, out_hbm.at[pl.ds(wid * BPW, BPW)])


def kernel(x, length, emb_weight):
    idx = x.astype(jnp.int32).reshape(2 * B, H)
    recip = jnp.broadcast_to((1.0 / length.astype(jnp.float32))[:, None],
                             (B, 16))
    return _bow_sc(emb_weight, idx, recip)


# speedup vs baseline: 3.2174x; 1.0029x over previous
import functools

import jax
import jax.numpy as jnp
from jax import lax
from jax.experimental import pallas as pl
from jax.experimental.pallas import tpu as pltpu
from jax.experimental.pallas import tpu_sc as plsc

B = 4096
L = 200
D = 64
H = 100
NW = 32
BPW = B // NW
NBUF = 4
NC = 2

_mesh = plsc.VectorSubcoreMesh(core_axis_name="c", subcore_axis_name="s")


@functools.partial(
    pl.kernel,
    mesh=_mesh,
    out_type=jax.ShapeDtypeStruct((B, D), jnp.float32),
    compiler_params=pltpu.CompilerParams(use_tc_tiling_on_sc=False),
    scratch_types=[
        pltpu.VMEM((2 * BPW, H), jnp.int32),
        pltpu.VMEM((BPW, 16), jnp.float32),
        pltpu.VMEM((NBUF, 2 * H, D), jnp.float32),
        pltpu.VMEM((BPW, D), jnp.float32),
        pltpu.SemaphoreType.DMA,
        pltpu.SemaphoreType.DMA,
        pltpu.SemaphoreType.DMA,
        pltpu.SemaphoreType.DMA,
    ],
)
def _bow_sc(table, idx_hbm, recip_hbm, out_hbm, idx_v, recip_v, bufs, out_v,
            sem0, sem1, sem2, sem3):
    sems = (sem0, sem1, sem2, sem3)
    wid = lax.axis_index("s") * NC + lax.axis_index("c")
    pltpu.sync_copy(idx_hbm.at[pl.ds(wid * (2 * BPW), 2 * BPW)], idx_v)
    pltpu.sync_copy(recip_hbm.at[pl.ds(wid * BPW, BPW)], recip_v)

    def fire(bag, k):
        pltpu.make_async_copy(
            table.at[idx_v.at[2 * bag]],
            bufs.at[k, pl.ds(0, H)], sems[k]).start()
        pltpu.make_async_copy(
            table.at[idx_v.at[2 * bag + 1]],
            bufs.at[k, pl.ds(H, H)], sems[k]).start()

    def wait(k):
        pltpu.make_async_copy(
            table.at[idx_v.at[0]], bufs.at[k, pl.ds(0, H)], sems[k]).wait()
        pltpu.make_async_copy(
            table.at[idx_v.at[0]], bufs.at[k, pl.ds(H, H)], sems[k]).wait()

    def reduce_bag(bag, k):
        def body(r, acc):
            a0, a1, a2, a3 = acc
            a0 = a0 + bufs[k, r, pl.ds(0, 16)]
            a1 = a1 + bufs[k, r, pl.ds(16, 16)]
            a2 = a2 + bufs[k, r, pl.ds(32, 16)]
            a3 = a3 + bufs[k, r, pl.ds(48, 16)]
            return a0, a1, a2, a3

        z = jnp.zeros((16,), jnp.float32)
        a0, a1, a2, a3 = lax.fori_loop(0, 2 * H, body, (z, z, z, z),
                                       unroll=25)
        rc = recip_v[bag, pl.ds(0, 16)]
        out_v[bag, pl.ds(0, 16)] = a0 * rc
        out_v[bag, pl.ds(16, 16)] = a1 * rc
        out_v[bag, pl.ds(32, 16)] = a2 * rc
        out_v[bag, pl.ds(48, 16)] = a3 * rc

    for k in range(NBUF - 1):
        fire(jnp.int32(k), k)

    def outer(g, carry):
        base = g * NBUF
        for k in range(NBUF):
            bag = base + k
            wait(k)
            nxt = bag + (NBUF - 1)

            @pl.when(nxt < BPW)
            def _():
                fire(nxt, (k + NBUF - 1) % NBUF)

            reduce_bag(bag, k)
        return carry

    lax.fori_loop(0, BPW // NBUF, outer, 0)
    pltpu.sync_copy(out_v, out_hbm.at[pl.ds(wid * BPW, BPW)])


def kernel(x, length, emb_weight):
    idx = x.astype(jnp.int32).reshape(2 * B, H)
    recip = jnp.broadcast_to((1.0 / length.astype(jnp.float32))[:, None],
                             (B, 16))
    return _bow_sc(emb_weight, idx, recip)
